# baseline (speedup 1.0000x reference)
import functools

import jax
import jax.numpy as jnp
from jax import lax
from jax.experimental import pallas as pl
from jax.experimental.pallas import tpu as pltpu
from jax.experimental.pallas import tpu_sc as plsc

VOCAB = 1000000
DIM = 64
ROWS = 16384 * 50
NUM_WORKERS = 32
NSPLIT = 2
RPART = ROWS // NSPLIT
NBUF = 4

_mesh = plsc.VectorSubcoreMesh(core_axis_name="c", subcore_axis_name="s")


def _make_gather(rows_part, chunk):
    per_w = rows_part // NUM_WORKERS
    nch = per_w // chunk
    nout = nch // NBUF
    assert per_w % chunk == 0 and nch % NBUF == 0

    @functools.partial(
        pl.kernel,
        mesh=_mesh,
        out_type=jax.ShapeDtypeStruct((rows_part, DIM), jnp.float32),
        scratch_types=[
            pltpu.VMEM((per_w,), jnp.int32),
            pltpu.VMEM((NBUF, chunk, DIM), jnp.float32),
            pltpu.SemaphoreType.DMA,
            pltpu.SemaphoreType.DMA,
            pltpu.SemaphoreType.DMA,
            pltpu.SemaphoreType.DMA,
            pltpu.SemaphoreType.DMA,
            pltpu.SemaphoreType.DMA,
            pltpu.SemaphoreType.DMA,
            pltpu.SemaphoreType.DMA,
        ],
        compiler_params=pltpu.CompilerParams(use_tc_tiling_on_sc=False),
    )
    def _gather(idx_hbm, table_hbm, out_hbm, idx_v, rows_v,
                g0, g1, g2, g3, w0, w1, w2, w3):
        gsem = (g0, g1, g2, g3)
        wsem = (w0, w1, w2, w3)
        wid = lax.axis_index("s") * 2 + lax.axis_index("c")
        base = wid * per_w
        pltpu.sync_copy(idx_hbm.at[pl.ds(base, per_w)], idx_v)

        def in_copy(off, b):
            return pltpu.make_async_copy(
                table_hbm.at[idx_v.at[pl.ds(off, chunk)]], rows_v.at[b], gsem[b])

        def out_copy(off, b):
            return pltpu.make_async_copy(
                rows_v.at[b], out_hbm.at[pl.ds(base + off, chunk)], wsem[b])

        for b in range(NBUF):
            in_copy(b * chunk, b).start()

        def body(g, carry):
            for b in range(NBUF):
                off = pl.multiple_of((g * NBUF + b) * chunk, chunk)
                in_copy(off, b).wait()
                out_copy(off, b).start()
                out_copy(off, b).wait()
                in_copy(off + NBUF * chunk, b).start()
            return carry

        lax.fori_loop(0, nout - 1, body, 0)

        for b in range(NBUF):
            off = ((nout - 1) * NBUF + b) * chunk
            in_copy(off, b).wait()
            out_copy(off, b).start()
        for b in range(NBUF):
            off = ((nout - 1) * NBUF + b) * chunk
            out_copy(off, b).wait()

    return _gather


_gather_part = _make_gather(RPART, 320)


def kernel(x, weight):
    idx = x.reshape(ROWS)
    parts = [
        _gather_part(idx[k * RPART:(k + 1) * RPART], weight)
        for k in range(NSPLIT)
    ]
    out = jnp.concatenate(parts, axis=0)
    return out.reshape(16384, 50, DIM)


# speedup vs baseline: 1.4029x; 1.4029x over previous
import functools

import jax
import jax.numpy as jnp
from jax import lax
from jax.experimental import pallas as pl
from jax.experimental.pallas import tpu as pltpu
from jax.experimental.pallas import tpu_sc as plsc

VOCAB = 1000000
DIM = 64
ROWS = 16384 * 50
NUM_WORKERS = 32
PER_W = ROWS // NUM_WORKERS
NBUF = 4
CHUNK = 320
NCH = PER_W // CHUNK
NOUT = NCH // NBUF

_mesh = plsc.VectorSubcoreMesh(core_axis_name="c", subcore_axis_name="s")


@functools.partial(
    pl.kernel,
    mesh=_mesh,
    out_type=jax.ShapeDtypeStruct((ROWS, 2 * DIM), jnp.float32),
    scratch_types=[
        pltpu.VMEM((PER_W,), jnp.int32),
        pltpu.VMEM((NBUF, CHUNK, DIM), jnp.float32),
        pltpu.SemaphoreType.DMA,
        pltpu.SemaphoreType.DMA,
        pltpu.SemaphoreType.DMA,
        pltpu.SemaphoreType.DMA,
        pltpu.SemaphoreType.DMA,
        pltpu.SemaphoreType.DMA,
        pltpu.SemaphoreType.DMA,
        pltpu.SemaphoreType.DMA,
    ],
    compiler_params=pltpu.CompilerParams(use_tc_tiling_on_sc=False),
)
def _gather(idx_hbm, table_hbm, out_hbm, idx_v, rows_v,
            g0, g1, g2, g3, w0, w1, w2, w3):
    gsem = (g0, g1, g2, g3)
    wsem = (w0, w1, w2, w3)
    wid = lax.axis_index("s") * 2 + lax.axis_index("c")
    base = wid * PER_W
    pltpu.sync_copy(idx_hbm.at[pl.ds(base, PER_W)], idx_v)

    def in_copy(off, b):
        return pltpu.make_async_copy(
            table_hbm.at[idx_v.at[pl.ds(off, CHUNK)]], rows_v.at[b], gsem[b])

    def out_copy(off, b):
        return pltpu.make_async_copy(
            rows_v.at[b],
            out_hbm.at[pl.ds(base + off, CHUNK), pl.ds(0, DIM)], wsem[b])

    for b in range(NBUF):
        in_copy(b * CHUNK, b).start()

    def body(g, carry):
        for b in range(NBUF):
            off = pl.multiple_of((g * NBUF + b) * CHUNK, CHUNK)
            in_copy(off, b).wait()
            out_copy(off, b).start()
            out_copy(off, b).wait()
            in_copy(off + NBUF * CHUNK, b).start()
        return carry

    lax.fori_loop(0, NOUT - 1, body, 0)

    for b in range(NBUF):
        off = ((NOUT - 1) * NBUF + b) * CHUNK
        in_copy(off, b).wait()
        out_copy(off, b).start()
    for b in range(NBUF):
        off = ((NOUT - 1) * NBUF + b) * CHUNK
        out_copy(off, b).wait()


def kernel(x, weight):
    idx = x.reshape(ROWS)
    out = _gather(idx, weight)
    return out[:, :DIM].reshape(16384, 50, DIM)


# speedup vs baseline: 1.5024x; 1.0709x over previous
import functools

import jax
import jax.numpy as jnp
from jax import lax
from jax.experimental import pallas as pl
from jax.experimental.pallas import tpu as pltpu
from jax.experimental.pallas import tpu_sc as plsc

VOCAB = 1000000
DIM = 64
ROWS = 16384 * 50
NUM_WORKERS = 32
PER_W = ROWS // NUM_WORKERS
NBUF = 4
CHUNK = 320
NCH = PER_W // CHUNK
NOUT = NCH // NBUF

_mesh = plsc.VectorSubcoreMesh(core_axis_name="c", subcore_axis_name="s")


@functools.partial(
    pl.kernel,
    mesh=_mesh,
    out_type=jax.ShapeDtypeStruct((ROWS, DIM), jnp.float32),
    scratch_types=[
        pltpu.VMEM((PER_W,), jnp.int32),
        pltpu.VMEM((NBUF, CHUNK, DIM), jnp.float32),
        pltpu.SemaphoreType.DMA,
        pltpu.SemaphoreType.DMA,
        pltpu.SemaphoreType.DMA,
        pltpu.SemaphoreType.DMA,
        pltpu.SemaphoreType.DMA,
        pltpu.SemaphoreType.DMA,
        pltpu.SemaphoreType.DMA,
        pltpu.SemaphoreType.DMA,
    ],
    compiler_params=pltpu.CompilerParams(use_tc_tiling_on_sc=False),
)
def _gather(idx_hbm, table_hbm, out_hbm, idx_v, rows_v,
            g0, g1, g2, g3, w0, w1, w2, w3):
    gsem = (g0, g1, g2, g3)
    wsem = (w0, w1, w2, w3)
    wid = lax.axis_index("s") * 2 + lax.axis_index("c")
    base = wid * PER_W
    pltpu.sync_copy(idx_hbm.at[pl.ds(base, PER_W)], idx_v)

    def in_copy(off, b):
        return pltpu.make_async_copy(
            table_hbm.at[idx_v.at[pl.ds(off, CHUNK)]], rows_v.at[b], gsem[b])

    def out_copy(off, b):
        return pltpu.make_async_copy(
            rows_v.at[b], out_hbm.at[pl.ds(base + off, CHUNK)], wsem[b])

    for b in range(NBUF):
        in_copy(b * CHUNK, b).start()

    def body(g, carry):
        for b in range(NBUF):
            off = pl.multiple_of((g * NBUF + b) * CHUNK, CHUNK)
            in_copy(off, b).wait()
            out_copy(off, b).start()
            out_copy(off, b).wait()
            in_copy(off + NBUF * CHUNK, b).start()
        return carry

    lax.fori_loop(0, NOUT - 1, body, 0)

    for b in range(NBUF):
        off = ((NOUT - 1) * NBUF + b) * CHUNK
        in_copy(off, b).wait()
        out_copy(off, b).start()
    for b in range(NBUF):
        off = ((NOUT - 1) * NBUF + b) * CHUNK
        out_copy(off, b).wait()


def kernel(x, weight):
    idx = x.reshape(ROWS)
    out = _gather(idx, weight)
    return out.reshape(16384, 50, DIM)


# speedup vs baseline: 1.5030x; 1.0004x over previous
import functools

import jax
import jax.numpy as jnp
from jax import lax
from jax.experimental import pallas as pl
from jax.experimental.pallas import tpu as pltpu
from jax.experimental.pallas import tpu_sc as plsc

VOCAB = 1000000
DIM = 64
ROWS = 16384 * 50
NUM_WORKERS = 32
PER_W = ROWS // NUM_WORKERS
NBUF = 2
CHUNK = 640
NCH = PER_W // CHUNK
NOUT = NCH // NBUF

_mesh = plsc.VectorSubcoreMesh(core_axis_name="c", subcore_axis_name="s")


@functools.partial(
    pl.kernel,
    mesh=_mesh,
    out_type=jax.ShapeDtypeStruct((ROWS, DIM), jnp.float32),
    scratch_types=[
        pltpu.VMEM((PER_W,), jnp.int32),
        pltpu.VMEM((NBUF, CHUNK, DIM), jnp.float32),
        pltpu.SemaphoreType.DMA,
        pltpu.SemaphoreType.DMA,
        pltpu.SemaphoreType.DMA,
        pltpu.SemaphoreType.DMA,
    ],
    compiler_params=pltpu.CompilerParams(use_tc_tiling_on_sc=False),
)
def _gather(idx_hbm, table_hbm, out_hbm, idx_v, rows_v, g0, g1, w0, w1):
    gsem = (g0, g1)
    wsem = (w0, w1)
    wid = lax.axis_index("s") * 2 + lax.axis_index("c")
    base = wid * PER_W
    pltpu.sync_copy(idx_hbm.at[pl.ds(base, PER_W)], idx_v)

    def in_copy(off, b):
        return pltpu.make_async_copy(
            table_hbm.at[idx_v.at[pl.ds(off, CHUNK)]], rows_v.at[b], gsem[b])

    def out_copy(off, b):
        return pltpu.make_async_copy(
            rows_v.at[b], out_hbm.at[pl.ds(base + off, CHUNK)], wsem[b])

    for b in range(NBUF):
        in_copy(b * CHUNK, b).start()

    def body(g, carry):
        for b in range(NBUF):
            off = pl.multiple_of((g * NBUF + b) * CHUNK, CHUNK)
            in_copy(off, b).wait()
            out_copy(off, b).start()
            out_copy(off, b).wait()
            in_copy(off + NBUF * CHUNK, b).start()
        return carry

    lax.fori_loop(0, NOUT - 1, body, 0)

    for b in range(NBUF):
        off = ((NOUT - 1) * NBUF + b) * CHUNK
        in_copy(off, b).wait()
        out_copy(off, b).start()
    for b in range(NBUF):
        off = ((NOUT - 1) * NBUF + b) * CHUNK
        out_copy(off, b).wait()


def kernel(x, weight):
    idx = x.reshape(ROWS)
    out = _gather(idx, weight)
    return out.reshape(16384, 50, DIM)
